# baseline (speedup 1.0000x reference)
import jax
import jax.numpy as jnp
from jax import lax
from jax.experimental import pallas as pl
from jax.experimental.pallas import tpu as pltpu


def _band(v, out_len, in_len):
    k = v.shape[-1]
    u = jnp.pad(v, [(0, 0)] * (v.ndim - 1) + [(0, in_len + 1 - k)])
    t = jnp.tile(u, (1,) * (v.ndim - 1) + (out_len,))
    t = t[..., : out_len * in_len]
    return t.reshape(v.shape[:-1] + (out_len, in_len))


def _net2_body(x_ref, w1b_ref, w2b_ref, wf1_ref, wf2_ref,
               b1_ref, b2_ref, bf1_ref, bf2_ref, out_ref, xs_ref):
    TB = x_ref.shape[0]
    f32 = jnp.float32
    bf16 = jnp.bfloat16

    xt = jnp.transpose(x_ref[:, 0], (1, 2, 0))
    xs_ref[:, :28, :] = xt
    xs_ref[:, 28:, :] = jnp.zeros((28, 4, TB), bf16)

    og = jnp.dot(w1b_ref[...], xs_ref[...].reshape(896, TB),
                 preferred_element_type=f32)
    v = og.reshape(12, 2, 96, TB)
    m = jnp.maximum(v[:, 0], v[:, 1]).reshape(12, 2, 48, TB)
    m = jnp.maximum(m[:, 0], m[:, 1])
    m = jnp.maximum(m + b1_ref[...], 0.0)
    p1 = m.reshape(576, TB).astype(bf16)

    o2 = jnp.dot(w2b_ref[...], p1, preferred_element_type=f32)
    v = o2.reshape(4, 2, 80, TB)
    m = jnp.maximum(v[:, 0], v[:, 1]).reshape(4, 2, 40, TB)
    m = jnp.maximum(m[:, 0], m[:, 1])
    p2 = jnp.maximum(m + b2_ref[...], 0.0).reshape(160, TB).astype(bf16)

    h1 = jnp.dot(wf1_ref[...], p2, preferred_element_type=f32) + bf1_ref[...]
    h1 = jnp.maximum(h1, 0.0).astype(bf16)
    z = jnp.dot(wf2_ref[...], h1, preferred_element_type=f32) + bf2_ref[...]

    zm = jnp.max(z, axis=0, keepdims=True)
    s = z - zm
    lse = jnp.log(jnp.sum(jnp.exp(s), axis=0, keepdims=True))
    out_ref[...] = (s - lse).T


@jax.jit
def _net2(w1s, b1s, w2s, b2s, wf1, bf1, wf2, bf2, x):
    B = x.shape[0]
    TB = 512 if B % 512 == 0 else (128 if B % 128 == 0 else B)
    f32 = jnp.float32
    bf16 = jnp.bfloat16

    w1r = w1s.reshape(4, 5, 5).astype(f32)
    a = _band(w1r, 24, 28)
    a = a.transpose(2, 0, 1, 3)
    a = a.reshape(12, 2, 4, 5, 28)
    a = a.transpose(1, 0, 2, 3, 4)
    base1 = jnp.pad(a, ((0, 0),) * 4 + ((0, 4),))
    base1 = base1.reshape(96, 160)
    w1b = jnp.concatenate(
        [jnp.pad(base1, ((0, 0), (32 * oy, 736 - 32 * oy)))
         for oy in range(24)], axis=0).astype(bf16)

    w2r = w2s.reshape(10, 4, 5, 5).astype(f32)
    c = _band(w2r, 8, 12)
    c = c.transpose(3, 0, 2, 4, 1)
    c = c.reshape(4, 2, 10, 5, 12, 4)
    c = c.transpose(1, 0, 2, 3, 4, 5)
    base2 = c.reshape(80, 240)
    w2b = jnp.concatenate(
        [jnp.pad(base2, ((0, 0), (48 * ry, 336 - 48 * ry)))
         for ry in range(8)], axis=0).astype(bf16)

    wf1p = wf1.reshape(100, 10, 4, 4).transpose(0, 2, 3, 1).reshape(100, 160)
    wf1p = wf1p.astype(bf16)
    wf2b = wf2.astype(bf16)

    b1bc = jnp.broadcast_to(jnp.tile(b1s.astype(f32), 12)[:, None], (48, TB))
    b2bc = jnp.broadcast_to(jnp.tile(b2s.astype(f32), 4)[:, None], (40, TB))
    bf1bc = jnp.broadcast_to(bf1.astype(f32).reshape(100, 1), (100, TB))
    bf2bc = jnp.broadcast_to(bf2.astype(f32).reshape(10, 1), (10, TB))

    out = pl.pallas_call(
        _net2_body,
        out_shape=jax.ShapeDtypeStruct((B, 10), f32),
        grid=(B // TB,),
        in_specs=[
            pl.BlockSpec((TB, 1, 28, 28), lambda i: (i, 0, 0, 0)),
            pl.BlockSpec((2304, 896), lambda i: (0, 0)),
            pl.BlockSpec((640, 576), lambda i: (0, 0)),
            pl.BlockSpec((100, 160), lambda i: (0, 0)),
            pl.BlockSpec((10, 100), lambda i: (0, 0)),
            pl.BlockSpec((48, TB), lambda i: (0, 0)),
            pl.BlockSpec((40, TB), lambda i: (0, 0)),
            pl.BlockSpec((100, TB), lambda i: (0, 0)),
            pl.BlockSpec((10, TB), lambda i: (0, 0)),
        ],
        out_specs=pl.BlockSpec((TB, 10), lambda i: (i, 0)),
        scratch_shapes=[pltpu.VMEM((28, 32, TB), jnp.bfloat16)],
        compiler_params=pltpu.CompilerParams(
            dimension_semantics=("parallel",)),
    )(x.astype(bf16), w1b, w2b, wf1p, wf2b, b1bc, b2bc, bf1bc, bf2bc)

    return out


def kernel(w1s, b1s, w2s, b2s, wf1, bf1, wf2, bf2, x):
    return _net2(w1s, b1s, w2s, b2s, wf1, bf1, wf2, bf2, x)


# speedup vs baseline: 1.2959x; 1.2959x over previous
import jax
import jax.numpy as jnp
from jax import lax
from jax.experimental import pallas as pl
from jax.experimental.pallas import tpu as pltpu


def _band(v, out_len, in_len):
    k = v.shape[-1]
    u = jnp.pad(v, [(0, 0)] * (v.ndim - 1) + [(0, in_len + 1 - k)])
    t = jnp.tile(u, (1,) * (v.ndim - 1) + (out_len,))
    t = t[..., : out_len * in_len]
    return t.reshape(v.shape[:-1] + (out_len, in_len))


def _net2_body(x_ref, w1b_ref, w2b_ref, wf1_ref, wf2_ref, b1_ref, out_ref):
    TB = x_ref.shape[0]
    f32 = jnp.float32
    bf16 = jnp.bfloat16

    og = lax.dot_general(w1b_ref[...], x_ref[...],
                         (((1,), (1,)), ((), ())),
                         preferred_element_type=f32)
    v = og.reshape(12, 2, 96, TB)
    m = jnp.maximum(v[:, 0], v[:, 1]).reshape(12, 2, 48, TB)
    m = jnp.maximum(m[:, 0], m[:, 1])
    m = jnp.maximum(m + b1_ref[...], 0.0)
    ones = jnp.ones((16, TB), bf16)
    p1 = jnp.concatenate([m.reshape(576, TB).astype(bf16), ones],
                         axis=0)

    o2 = jnp.dot(w2b_ref[...], p1, preferred_element_type=f32)
    v = o2.reshape(4, 2, 80, TB)
    m = jnp.maximum(v[:, 0], v[:, 1]).reshape(4, 2, 40, TB)
    m = jnp.maximum(m[:, 0], m[:, 1])
    p2 = jnp.concatenate([jnp.maximum(m, 0.0).reshape(160, TB).astype(bf16),
                          ones], axis=0)

    h1 = jnp.maximum(jnp.dot(wf1_ref[...], p2, preferred_element_type=f32),
                     0.0)
    h1 = jnp.concatenate([h1.astype(bf16), ones], axis=0)
    z = jnp.dot(wf2_ref[...], h1, preferred_element_type=f32)

    zm = jnp.max(z, axis=0, keepdims=True)
    s = z - zm
    lse = jnp.log(jnp.sum(jnp.exp(s), axis=0, keepdims=True))
    out_ref[...] = (s - lse).T


@jax.jit
def _net2(w1s, b1s, w2s, b2s, wf1, bf1, wf2, bf2, x):
    B = x.shape[0]
    TB = 1024 if B % 1024 == 0 else (128 if B % 128 == 0 else B)
    f32 = jnp.float32
    bf16 = jnp.bfloat16

    w1r = w1s.reshape(4, 5, 5).astype(f32)
    a = _band(w1r, 24, 28)
    a = a.transpose(2, 0, 1, 3)
    a = a.reshape(12, 2, 4, 5, 28)
    a = a.transpose(1, 0, 2, 3, 4)
    base1 = a.reshape(96, 140)
    w1b = jnp.concatenate(
        [jnp.pad(base1, ((0, 0), (28 * oy, 644 - 28 * oy)))
         for oy in range(24)], axis=0).astype(bf16)

    w2r = w2s.reshape(10, 4, 5, 5).astype(f32)
    c = _band(w2r, 8, 12)
    c = c.transpose(3, 0, 2, 4, 1)
    c = c.reshape(4, 2, 10, 5, 12, 4)
    c = c.transpose(1, 0, 2, 3, 4, 5)
    base2 = c.reshape(80, 240)
    b2row = jnp.tile(b2s.astype(f32), 8)[:, None]
    w2b = jnp.concatenate(
        [jnp.concatenate(
            [jnp.pad(base2, ((0, 0), (48 * ry, 336 - 48 * ry))),
             b2row, jnp.zeros((80, 15), f32)], axis=1)
         for ry in range(8)], axis=0).astype(bf16)

    wf1p = wf1.reshape(100, 10, 4, 4).transpose(0, 2, 3, 1).reshape(100, 160)
    wf1p = jnp.concatenate(
        [wf1p, bf1.reshape(100, 1), jnp.zeros((100, 15), f32)],
        axis=1).astype(bf16)
    wf2b = jnp.concatenate(
        [wf2, bf2.reshape(10, 1), jnp.zeros((10, 15), f32)],
        axis=1).astype(bf16)

    b1bc = jnp.broadcast_to(jnp.tile(b1s.astype(f32), 12)[:, None], (48, TB))

    xb = x.reshape(B, 784).astype(bf16)

    out = pl.pallas_call(
        _net2_body,
        out_shape=jax.ShapeDtypeStruct((B, 10), f32),
        grid=(B // TB,),
        in_specs=[
            pl.BlockSpec((TB, 784), lambda i: (i, 0)),
            pl.BlockSpec((2304, 784), lambda i: (0, 0)),
            pl.BlockSpec((640, 592), lambda i: (0, 0)),
            pl.BlockSpec((100, 176), lambda i: (0, 0)),
            pl.BlockSpec((10, 116), lambda i: (0, 0)),
            pl.BlockSpec((48, TB), lambda i: (0, 0)),
        ],
        out_specs=pl.BlockSpec((TB, 10), lambda i: (i, 0)),
        compiler_params=pltpu.CompilerParams(
            dimension_semantics=("parallel",)),
    )(xb, w1b, w2b, wf1p, wf2b, b1bc)

    return out


def kernel(w1s, b1s, w2s, b2s, wf1, bf1, wf2, bf2, x):
    return _net2(w1s, b1s, w2s, b2s, wf1, bf1, wf2, bf2, x)


# speedup vs baseline: 1.3072x; 1.0087x over previous
import jax
import jax.numpy as jnp
from jax import lax
from jax.experimental import pallas as pl
from jax.experimental.pallas import tpu as pltpu


def _band(v, out_len, in_len):
    k = v.shape[-1]
    u = jnp.pad(v, [(0, 0)] * (v.ndim - 1) + [(0, in_len + 1 - k)])
    t = jnp.tile(u, (1,) * (v.ndim - 1) + (out_len,))
    t = t[..., : out_len * in_len]
    return t.reshape(v.shape[:-1] + (out_len, in_len))


def _net2_body(x_ref, w1b_ref, w2b_ref, wf1_ref, wf2_ref, b1_ref, out_ref):
    TB = x_ref.shape[0]
    f32 = jnp.float32
    bf16 = jnp.bfloat16

    og = lax.dot_general(w1b_ref[...], x_ref[...],
                         (((1,), (1,)), ((), ())),
                         preferred_element_type=f32)
    v = og.reshape(12, 2, 96, TB)
    m = jnp.maximum(v[:, 0], v[:, 1]).reshape(12, 2, 48, TB)
    m = jnp.maximum(m[:, 0], m[:, 1])
    m = jnp.maximum(m + b1_ref[...], 0.0)
    ones = jnp.ones((16, TB), bf16)
    p1 = jnp.concatenate([m.reshape(576, TB).astype(bf16), ones],
                         axis=0)

    o2 = jnp.dot(w2b_ref[...], p1, preferred_element_type=f32)
    v = o2.reshape(4, 2, 80, TB)
    m = jnp.maximum(v[:, 0], v[:, 1]).reshape(4, 2, 40, TB)
    m = jnp.maximum(m[:, 0], m[:, 1])
    p2 = jnp.concatenate([jnp.maximum(m, 0.0).reshape(160, TB).astype(bf16),
                          ones], axis=0)

    h1 = jnp.maximum(jnp.dot(wf1_ref[...], p2, preferred_element_type=f32),
                     0.0)
    h1 = jnp.concatenate([h1.astype(bf16), ones], axis=0)
    z = jnp.dot(wf2_ref[...], h1, preferred_element_type=f32)

    zm = jnp.max(z, axis=0, keepdims=True)
    s = z - zm
    lse = jnp.log(jnp.sum(jnp.exp(s), axis=0, keepdims=True))
    out_ref[...] = (s - lse).T


@jax.jit
def _net2(w1s, b1s, w2s, b2s, wf1, bf1, wf2, bf2, x):
    B = x.shape[0]
    TB = 2048 if B % 2048 == 0 else (128 if B % 128 == 0 else B)
    f32 = jnp.float32
    bf16 = jnp.bfloat16

    w1r = w1s.reshape(4, 5, 5).astype(f32)
    a = _band(w1r, 24, 28)
    a = a.transpose(2, 0, 1, 3)
    a = a.reshape(12, 2, 4, 5, 28)
    a = a.transpose(1, 0, 2, 3, 4)
    base1 = a.reshape(96, 140)
    w1b = jnp.concatenate(
        [jnp.pad(base1, ((0, 0), (28 * oy, 644 - 28 * oy)))
         for oy in range(24)], axis=0).astype(bf16)

    w2r = w2s.reshape(10, 4, 5, 5).astype(f32)
    c = _band(w2r, 8, 12)
    c = c.transpose(3, 0, 2, 4, 1)
    c = c.reshape(4, 2, 10, 5, 12, 4)
    c = c.transpose(1, 0, 2, 3, 4, 5)
    base2 = c.reshape(80, 240)
    b2row = jnp.tile(b2s.astype(f32), 8)[:, None]
    w2b = jnp.concatenate(
        [jnp.concatenate(
            [jnp.pad(base2, ((0, 0), (48 * ry, 336 - 48 * ry))),
             b2row, jnp.zeros((80, 15), f32)], axis=1)
         for ry in range(8)], axis=0).astype(bf16)

    wf1p = wf1.reshape(100, 10, 4, 4).transpose(0, 2, 3, 1).reshape(100, 160)
    wf1p = jnp.concatenate(
        [wf1p, bf1.reshape(100, 1), jnp.zeros((100, 15), f32)],
        axis=1).astype(bf16)
    wf2b = jnp.concatenate(
        [wf2, bf2.reshape(10, 1), jnp.zeros((10, 15), f32)],
        axis=1).astype(bf16)

    b1bc = jnp.broadcast_to(jnp.tile(b1s.astype(f32), 12)[:, None], (48, TB))

    xb = x.reshape(B, 784).astype(bf16)

    out = pl.pallas_call(
        _net2_body,
        out_shape=jax.ShapeDtypeStruct((B, 10), f32),
        grid=(B // TB,),
        in_specs=[
            pl.BlockSpec((TB, 784), lambda i: (i, 0)),
            pl.BlockSpec((2304, 784), lambda i: (0, 0)),
            pl.BlockSpec((640, 592), lambda i: (0, 0)),
            pl.BlockSpec((100, 176), lambda i: (0, 0)),
            pl.BlockSpec((10, 116), lambda i: (0, 0)),
            pl.BlockSpec((48, TB), lambda i: (0, 0)),
        ],
        out_specs=pl.BlockSpec((TB, 10), lambda i: (i, 0)),
        compiler_params=pltpu.CompilerParams(
            dimension_semantics=("parallel",)),
    )(xb, w1b, w2b, wf1p, wf2b, b1bc)

    return out


def kernel(w1s, b1s, w2s, b2s, wf1, bf1, wf2, bf2, x):
    return _net2(w1s, b1s, w2s, b2s, wf1, bf1, wf2, bf2, x)


# speedup vs baseline: 1.6664x; 1.2748x over previous
import jax
import jax.numpy as jnp
from jax import lax
from jax.experimental import pallas as pl
from jax.experimental.pallas import tpu as pltpu


def _band(v, out_len, in_len):
    k = v.shape[-1]
    u = jnp.pad(v, [(0, 0)] * (v.ndim - 1) + [(0, in_len + 1 - k)])
    t = jnp.tile(u, (1,) * (v.ndim - 1) + (out_len,))
    t = t[..., : out_len * in_len]
    return t.reshape(v.shape[:-1] + (out_len, in_len))


def _net2_body(x_ref, w1b_ref, w2b_ref, wf1_ref, wf2_ref, b1_ref, out_ref,
               xs_ref):
    TB = x_ref.shape[0]
    f32 = jnp.float32
    bf16 = jnp.bfloat16

    xt = jnp.transpose(x_ref[...].astype(bf16), (1, 2, 0))
    xs_ref[:, :28, :] = xt
    xs_ref[:, 28:, :] = jnp.zeros((28, 4, TB), bf16)

    og = jnp.dot(w1b_ref[...], xs_ref[...].reshape(896, TB),
                 preferred_element_type=f32)
    v = og.reshape(12, 2, 96, TB)
    m = jnp.maximum(v[:, 0], v[:, 1]).reshape(12, 2, 48, TB)
    m = jnp.maximum(m[:, 0], m[:, 1])
    m = jnp.maximum(m + b1_ref[...], 0.0)
    ones = jnp.ones((16, TB), bf16)
    p1 = jnp.concatenate([m.reshape(576, TB).astype(bf16), ones],
                         axis=0)

    o2 = jnp.dot(w2b_ref[...], p1, preferred_element_type=f32)
    v = o2.reshape(4, 2, 80, TB)
    m = jnp.maximum(v[:, 0], v[:, 1]).reshape(4, 2, 40, TB)
    m = jnp.maximum(m[:, 0], m[:, 1])
    p2 = jnp.concatenate([jnp.maximum(m, 0.0).reshape(160, TB).astype(bf16),
                          ones], axis=0)

    h1 = jnp.maximum(jnp.dot(wf1_ref[...], p2, preferred_element_type=f32),
                     0.0)
    h1 = jnp.concatenate([h1.astype(bf16), ones], axis=0)
    z = jnp.dot(wf2_ref[...], h1, preferred_element_type=f32)

    zm = jnp.max(z, axis=0, keepdims=True)
    s = z - zm
    lse = jnp.log(jnp.sum(jnp.exp(s), axis=0, keepdims=True))
    out_ref[...] = (s - lse).T


@jax.jit
def _net2(w1s, b1s, w2s, b2s, wf1, bf1, wf2, bf2, x):
    B = x.shape[0]
    TB = 512 if B % 512 == 0 else (128 if B % 128 == 0 else B)
    f32 = jnp.float32
    bf16 = jnp.bfloat16

    w1r = w1s.reshape(4, 5, 5).astype(f32)
    a = _band(w1r, 24, 28)
    a = a.transpose(2, 0, 1, 3)
    a = a.reshape(12, 2, 4, 5, 28)
    a = a.transpose(1, 0, 2, 3, 4)
    base1 = jnp.pad(a, ((0, 0),) * 4 + ((0, 4),))
    base1 = base1.reshape(96, 160)
    w1b = jnp.concatenate(
        [jnp.pad(base1, ((0, 0), (32 * oy, 736 - 32 * oy)))
         for oy in range(24)], axis=0).astype(bf16)

    w2r = w2s.reshape(10, 4, 5, 5).astype(f32)
    c = _band(w2r, 8, 12)
    c = c.transpose(3, 0, 2, 4, 1)
    c = c.reshape(4, 2, 10, 5, 12, 4)
    c = c.transpose(1, 0, 2, 3, 4, 5)
    base2 = c.reshape(80, 240)
    b2row = jnp.tile(b2s.astype(f32), 8)[:, None]
    w2b = jnp.concatenate(
        [jnp.concatenate(
            [jnp.pad(base2, ((0, 0), (48 * ry, 336 - 48 * ry))),
             b2row, jnp.zeros((80, 15), f32)], axis=1)
         for ry in range(8)], axis=0).astype(bf16)

    wf1p = wf1.reshape(100, 10, 4, 4).transpose(0, 2, 3, 1).reshape(100, 160)
    wf1p = jnp.concatenate(
        [wf1p, bf1.reshape(100, 1), jnp.zeros((100, 15), f32)],
        axis=1).astype(bf16)
    wf2b = jnp.concatenate(
        [wf2, bf2.reshape(10, 1), jnp.zeros((10, 15), f32)],
        axis=1).astype(bf16)

    b1bc = jnp.broadcast_to(jnp.tile(b1s.astype(f32), 12)[:, None], (48, TB))

    xb = x.reshape(B, 28, 28)

    out = pl.pallas_call(
        _net2_body,
        out_shape=jax.ShapeDtypeStruct((B, 10), f32),
        grid=(B // TB,),
        in_specs=[
            pl.BlockSpec((TB, 28, 28), lambda i: (i, 0, 0)),
            pl.BlockSpec((2304, 896), lambda i: (0, 0)),
            pl.BlockSpec((640, 592), lambda i: (0, 0)),
            pl.BlockSpec((100, 176), lambda i: (0, 0)),
            pl.BlockSpec((10, 116), lambda i: (0, 0)),
            pl.BlockSpec((48, TB), lambda i: (0, 0)),
        ],
        out_specs=pl.BlockSpec((TB, 10), lambda i: (i, 0)),
        scratch_shapes=[pltpu.VMEM((28, 32, TB), jnp.bfloat16)],
        compiler_params=pltpu.CompilerParams(
            dimension_semantics=("parallel",)),
    )(xb, w1b, w2b, wf1p, wf2b, b1bc)

    return out


def kernel(w1s, b1s, w2s, b2s, wf1, bf1, wf2, bf2, x):
    return _net2(w1s, b1s, w2s, b2s, wf1, bf1, wf2, bf2, x)


# speedup vs baseline: 1.8641x; 1.1187x over previous
import jax
import jax.numpy as jnp
from jax import lax
from jax.experimental import pallas as pl
from jax.experimental.pallas import tpu as pltpu


def _band(v, out_len, in_len):
    k = v.shape[-1]
    u = jnp.pad(v, [(0, 0)] * (v.ndim - 1) + [(0, in_len + 1 - k)])
    t = jnp.tile(u, (1,) * (v.ndim - 1) + (out_len,))
    t = t[..., : out_len * in_len]
    return t.reshape(v.shape[:-1] + (out_len, in_len))


def _net2_body(x_ref, w1b_ref, w2b_ref, wf1_ref, wf2_ref, b1_ref, out_ref,
               xs_ref):
    TB = x_ref.shape[0]
    f32 = jnp.float32
    bf16 = jnp.bfloat16

    xt = jnp.transpose(x_ref[...].astype(bf16), (1, 2, 0))
    xs_ref[:, :28, :] = xt
    xs_ref[:, 28:, :] = jnp.zeros((28, 4, TB), bf16)

    og = jnp.dot(w1b_ref[...], xs_ref[...].reshape(896, TB),
                 preferred_element_type=f32)
    v = og.reshape(12, 2, 96, TB)
    m = jnp.maximum(v[:, 0], v[:, 1]).reshape(12, 2, 48, TB)
    m = jnp.maximum(m[:, 0], m[:, 1])
    m = jnp.maximum(m + b1_ref[...], 0.0)
    ones = jnp.ones((16, TB), bf16)
    p1 = jnp.concatenate([m.reshape(576, TB).astype(bf16), ones],
                         axis=0)

    o2 = jnp.dot(w2b_ref[...], p1, preferred_element_type=f32)
    v = o2.reshape(4, 2, 80, TB)
    m = jnp.maximum(v[:, 0], v[:, 1]).reshape(4, 2, 40, TB)
    m = jnp.maximum(m[:, 0], m[:, 1])
    p2 = jnp.concatenate([jnp.maximum(m, 0.0).reshape(160, TB).astype(bf16),
                          ones], axis=0)

    h1 = jnp.maximum(jnp.dot(wf1_ref[...], p2, preferred_element_type=f32),
                     0.0)
    h1 = jnp.concatenate([h1.astype(bf16), ones], axis=0)
    z = jnp.dot(wf2_ref[...], h1, preferred_element_type=f32)

    zm = jnp.max(z, axis=0, keepdims=True)
    s = z - zm
    lse = jnp.log(jnp.sum(jnp.exp(s), axis=0, keepdims=True))
    out_ref[...] = (s - lse).T


@jax.jit
def _net2(w1s, b1s, w2s, b2s, wf1, bf1, wf2, bf2, x):
    B = x.shape[0]
    TB = 1024 if B % 1024 == 0 else (128 if B % 128 == 0 else B)
    f32 = jnp.float32
    bf16 = jnp.bfloat16

    w1r = w1s.reshape(4, 5, 5).astype(f32)
    a = _band(w1r, 24, 28)
    a = a.transpose(2, 0, 1, 3)
    a = a.reshape(12, 2, 4, 5, 28)
    a = a.transpose(1, 0, 2, 3, 4)
    base1 = jnp.pad(a, ((0, 0),) * 4 + ((0, 4),))
    base1 = base1.reshape(96, 160)
    w1b = jnp.concatenate(
        [jnp.pad(base1, ((0, 0), (32 * oy, 736 - 32 * oy)))
         for oy in range(24)], axis=0).astype(bf16)

    w2r = w2s.reshape(10, 4, 5, 5).astype(f32)
    c = _band(w2r, 8, 12)
    c = c.transpose(3, 0, 2, 4, 1)
    c = c.reshape(4, 2, 10, 5, 12, 4)
    c = c.transpose(1, 0, 2, 3, 4, 5)
    base2 = c.reshape(80, 240)
    b2row = jnp.tile(b2s.astype(f32), 8)[:, None]
    w2b = jnp.concatenate(
        [jnp.concatenate(
            [jnp.pad(base2, ((0, 0), (48 * ry, 336 - 48 * ry))),
             b2row, jnp.zeros((80, 15), f32)], axis=1)
         for ry in range(8)], axis=0).astype(bf16)

    wf1p = wf1.reshape(100, 10, 4, 4).transpose(0, 2, 3, 1).reshape(100, 160)
    wf1p = jnp.concatenate(
        [wf1p, bf1.reshape(100, 1), jnp.zeros((100, 15), f32)],
        axis=1).astype(bf16)
    wf2b = jnp.concatenate(
        [wf2, bf2.reshape(10, 1), jnp.zeros((10, 15), f32)],
        axis=1).astype(bf16)

    b1bc = jnp.broadcast_to(jnp.tile(b1s.astype(f32), 12)[:, None], (48, TB))

    xb = x.reshape(B, 28, 28)

    out = pl.pallas_call(
        _net2_body,
        out_shape=jax.ShapeDtypeStruct((B, 10), f32),
        grid=(B // TB,),
        in_specs=[
            pl.BlockSpec((TB, 28, 28), lambda i: (i, 0, 0)),
            pl.BlockSpec((2304, 896), lambda i: (0, 0)),
            pl.BlockSpec((640, 592), lambda i: (0, 0)),
            pl.BlockSpec((100, 176), lambda i: (0, 0)),
            pl.BlockSpec((10, 116), lambda i: (0, 0)),
            pl.BlockSpec((48, TB), lambda i: (0, 0)),
        ],
        out_specs=pl.BlockSpec((TB, 10), lambda i: (i, 0)),
        scratch_shapes=[pltpu.VMEM((28, 32, TB), jnp.bfloat16)],
        compiler_params=pltpu.CompilerParams(
            dimension_semantics=("parallel",)),
    )(xb, w1b, w2b, wf1p, wf2b, b1bc)

    return out


def kernel(w1s, b1s, w2s, b2s, wf1, bf1, wf2, bf2, x):
    return _net2(w1s, b1s, w2s, b2s, wf1, bf1, wf2, bf2, x)
